# baseline (speedup 1.0000x reference)
import jax
import jax.numpy as jnp
from jax import lax
from jax.experimental import pallas as pl
from jax.experimental.pallas import tpu as pltpu
from jax.experimental.pallas import tpu_sc as plsc

B = 16384
F = 26
K = 16
FP = 32
EMBP = FP * K
NC, NS = 2, 16
NW = NC * NS
TOTAL = B * F
TOTE = B * FP
CHUNK = 128
ECHUNKS = TOTE // CHUNK // NW
FCHUNKS = TOTAL // CHUNK // NW
KBE = 16
NGRP = ECHUNKS // KBE
KBF = FCHUNKS // NGRP
GROWS = KBE * CHUNK
PER_WE = ECHUNKS * CHUNK


def _sc_gather_body(xe_hbm, xf_hbm, emb_hbm, fc_hbm, rows_out, fc_out,
                    idxe_v, idxf_v, rows2, fcall, esem0, esem1, fsem,
                    osem0, osem1):
    wid = lax.axis_index("s") * NC + lax.axis_index("c")

    pltpu.sync_copy(xe_hbm.at[pl.ds(wid * ECHUNKS, ECHUNKS)], idxe_v)
    pltpu.sync_copy(xf_hbm.at[pl.ds(wid * FCHUNKS, FCHUNKS)], idxf_v)

    esem = (esem0, esem1)
    osem = (osem0, osem1)
    edesc = [None] * NGRP
    odesc = [None] * NGRP

    def fire_group(g):
        slot = g & 1
        ds_ = []
        for j in range(KBE):
            ds_.append(pltpu.async_copy(
                emb_hbm.at[idxe_v.at[g * KBE + j]],
                rows2.at[slot].at[pl.ds(j * CHUNK, CHUNK)],
                esem[slot]))
        for j in range(KBF):
            c = g * KBF + j
            pltpu.async_copy(fc_hbm.at[idxf_v.at[c]], fcall.at[c], fsem)
        edesc[g] = ds_

    def drain_and_writeback(g):
        slot = g & 1
        for d in edesc[g]:
            d.wait()
        odesc[g] = pltpu.async_copy(
            rows2.at[slot],
            rows_out.at[pl.ds(wid * PER_WE + g * GROWS, GROWS)],
            osem[slot])

    for g in range(NGRP):
        if g >= 2:
            odesc[g - 2].wait()
        fire_group(g)
        if g >= 1:
            drain_and_writeback(g - 1)
    drain_and_writeback(NGRP - 1)
    odesc[NGRP - 2].wait()
    odesc[NGRP - 1].wait()

    pltpu.make_async_copy(fc_out.at[pl.ds(0, FCHUNKS)], fcall, fsem).wait()
    pltpu.sync_copy(fcall, fc_out.at[pl.ds(wid * FCHUNKS, FCHUNKS)])


def _sc_gather(xperm, xflat, emb_table, fc_flat):
    mesh = plsc.VectorSubcoreMesh(
        core_axis_name="c", subcore_axis_name="s",
        num_cores=NC, num_subcores=NS)
    return pl.kernel(
        _sc_gather_body,
        out_type=[
            jax.ShapeDtypeStruct((TOTE, K), jnp.float32),
            jax.ShapeDtypeStruct((TOTAL // CHUNK, CHUNK), jnp.float32),
        ],
        mesh=mesh,
        scratch_types=[
            pltpu.VMEM((ECHUNKS, CHUNK), jnp.int32),
            pltpu.VMEM((FCHUNKS, CHUNK), jnp.int32),
            pltpu.VMEM((2, GROWS, K), jnp.float32),
            pltpu.VMEM((FCHUNKS, CHUNK), jnp.float32),
            pltpu.SemaphoreType.DMA,
            pltpu.SemaphoreType.DMA,
            pltpu.SemaphoreType.DMA,
            pltpu.SemaphoreType.DMA,
            pltpu.SemaphoreType.DMA,
        ],
        compiler_params=pltpu.CompilerParams(use_tc_tiling_on_sc=False),
    )(xperm, xflat, emb_table, fc_flat)


BLK = 2048
SROWS = BLK * EMBP // 128


def _dense_body(h_ref, fc_ref, linw_ref, linb_ref, w1_ref, b1_ref,
                w2_ref, b2_ref, w3_ref, b3_ref, y_ref):
    s4 = h_ref[...].reshape(BLK, 4, 128)

    s = jnp.zeros((BLK, K), jnp.float32)
    sq = jnp.zeros((BLK, K), jnp.float32)
    a1 = jnp.zeros((BLK, 128), jnp.float32)
    c = lax.broadcasted_iota(jnp.int32, (128, K), 0)
    k = lax.broadcasted_iota(jnp.int32, (128, K), 1)
    match = (c % K) == k
    for ct in range(4):
        valid = (ct * 8 + c // K) < F
        sel = jnp.where(match & valid, 1.0, 0.0)
        hct = s4[:, ct]
        s = s + jnp.dot(hct, sel, preferred_element_type=jnp.float32)
        sq = sq + jnp.dot(hct * hct, sel, preferred_element_type=jnp.float32)
        a1 = a1 + jnp.dot(hct, w1_ref[pl.ds(ct * 128, 128)],
                          preferred_element_type=jnp.float32)
    inter = jnp.sum(s * s - sq, axis=1, keepdims=True)

    tot = jnp.sum(fc_ref[...])
    linw = jnp.sum(linw_ref[...])
    linb = jnp.sum(linb_ref[...])
    linear_term = linw * tot + jnp.float32(TOTAL) * linb

    h1 = jnp.maximum(a1 + b1_ref[...], 0.0)
    h2 = jnp.maximum(
        jnp.dot(h1, w2_ref[...], preferred_element_type=jnp.float32)
        + b2_ref[...], 0.0)
    m = jnp.dot(h2, w3_ref[...], preferred_element_type=jnp.float32) + b3_ref[...]

    z = linear_term + 0.5 * inter + m
    y_ref[...] = 1.0 / (1.0 + jnp.exp(-z))


def _dense(hslab, fcvals, lin_W, lin_b, W1p, b1, W2, b2, W3, b3):
    grid = (B // BLK,)
    zero = lambda i: (0, 0)
    return pl.pallas_call(
        _dense_body,
        out_shape=jax.ShapeDtypeStruct((B, 1), jnp.float32),
        grid=grid,
        in_specs=[
            pl.BlockSpec((SROWS, 128), lambda i: (i, 0)),
            pl.BlockSpec((TOTAL // CHUNK, CHUNK), zero),
            pl.BlockSpec((1, 1), zero),
            pl.BlockSpec((1, 1), zero),
            pl.BlockSpec((EMBP, 128), zero),
            pl.BlockSpec((1, 128), zero),
            pl.BlockSpec((128, 64), zero),
            pl.BlockSpec((1, 64), zero),
            pl.BlockSpec((64, 1), zero),
            pl.BlockSpec((1, 1), zero),
        ],
        out_specs=pl.BlockSpec((BLK, 1), lambda i: (i, 0)),
    )(hslab, fcvals, lin_W, lin_b, W1p, b1, W2, b2, W3, b3)


def kernel(x, fc_table, emb_table, lin_W, lin_b, W1, b1, W2, b2, W3, b3):
    xp = jnp.concatenate([x, x[:, :FP - F]], axis=1)
    xperm = xp.reshape(TOTE // CHUNK, CHUNK)
    xflat = x.reshape(TOTAL // CHUNK, CHUNK)
    fc_flat = fc_table.reshape(-1)
    W1p = jnp.pad(W1, ((0, EMBP - F * K), (0, 0)))
    rows, fcvals = _sc_gather(xperm, xflat, emb_table, fc_flat)
    hslab = rows.reshape(TOTE * K // 128, 128)
    return _dense(hslab, fcvals, lin_W, lin_b.reshape(1, 1),
                  W1p, b1.reshape(1, -1), W2, b2.reshape(1, -1),
                  W3, b3.reshape(1, 1))


# speedup vs baseline: 1.2350x; 1.2350x over previous
import jax
import jax.numpy as jnp
from jax import lax
from jax.experimental import pallas as pl
from jax.experimental.pallas import tpu as pltpu
from jax.experimental.pallas import tpu_sc as plsc

B = 16384
F = 26
K = 16
FP = 32
EMBP = FP * K
NC, NS = 2, 16
NW = NC * NS
TOTAL = B * F
TOTE = B * FP
CHUNK = 128
ECHUNKS = TOTE // CHUNK // NW
FCHUNKS = TOTAL // CHUNK // NW
KBE = 16
NGRP = ECHUNKS // KBE
KBF = FCHUNKS // NGRP
GROWS = KBE * CHUNK
PER_WE = ECHUNKS * CHUNK


def _sc_gather_body(xe_hbm, xf_hbm, emb_hbm, fc_hbm, rows_out, fc_out,
                    idxe_v, idxf_v, rows2, fcall, esem0, esem1, fsem,
                    osem0, osem1):
    wid = lax.axis_index("s") * NC + lax.axis_index("c")

    pltpu.sync_copy(xe_hbm.at[pl.ds(wid * ECHUNKS, ECHUNKS)], idxe_v)
    pltpu.sync_copy(xf_hbm.at[pl.ds(wid * FCHUNKS, FCHUNKS)], idxf_v)

    esem = (esem0, esem1)
    osem = (osem0, osem1)
    edesc = [None] * NGRP
    odesc = [None] * NGRP

    def fire_group(g):
        slot = g & 1
        ds_ = []
        for j in range(KBE):
            ds_.append(pltpu.async_copy(
                emb_hbm.at[idxe_v.at[g * KBE + j]],
                rows2.at[slot].at[pl.ds(j * CHUNK, CHUNK)],
                esem[slot]))
        for j in range(KBF):
            c = g * KBF + j
            pltpu.async_copy(fc_hbm.at[idxf_v.at[c]], fcall.at[c], fsem)
        edesc[g] = ds_

    def drain_and_writeback(g):
        slot = g & 1
        for d in edesc[g]:
            d.wait()
        odesc[g] = pltpu.async_copy(
            rows2.at[slot],
            rows_out.at[pl.ds(wid * PER_WE + g * GROWS, GROWS)],
            osem[slot])

    for g in range(NGRP):
        if g >= 2:
            odesc[g - 2].wait()
        fire_group(g)
        if g >= 1:
            drain_and_writeback(g - 1)
    drain_and_writeback(NGRP - 1)
    odesc[NGRP - 2].wait()
    odesc[NGRP - 1].wait()

    pltpu.make_async_copy(fc_out.at[pl.ds(0, FCHUNKS)], fcall, fsem).wait()
    pltpu.sync_copy(fcall, fc_out.at[pl.ds(wid * FCHUNKS, FCHUNKS)])


def _sc_gather(xperm, xflat, emb_table, fc_flat):
    mesh = plsc.VectorSubcoreMesh(
        core_axis_name="c", subcore_axis_name="s",
        num_cores=NC, num_subcores=NS)
    return pl.kernel(
        _sc_gather_body,
        out_type=[
            jax.ShapeDtypeStruct((TOTE, K), jnp.float32),
            jax.ShapeDtypeStruct((TOTAL // CHUNK, CHUNK), jnp.float32),
        ],
        mesh=mesh,
        scratch_types=[
            pltpu.VMEM((ECHUNKS, CHUNK), jnp.int32),
            pltpu.VMEM((FCHUNKS, CHUNK), jnp.int32),
            pltpu.VMEM((2, GROWS, K), jnp.float32),
            pltpu.VMEM((FCHUNKS, CHUNK), jnp.float32),
            pltpu.SemaphoreType.DMA,
            pltpu.SemaphoreType.DMA,
            pltpu.SemaphoreType.DMA,
            pltpu.SemaphoreType.DMA,
            pltpu.SemaphoreType.DMA,
        ],
        compiler_params=pltpu.CompilerParams(use_tc_tiling_on_sc=False),
    )(xperm, xflat, emb_table, fc_flat)


CW = 8192
NTB = (1000012 + CW - 1) // CW
VPAD = NTB * CW


def _tr_body(t_ref, o_ref):
    a = t_ref[...]
    a3 = jnp.transpose(a).reshape(CW // 8, 8, K)
    for j in range(8):
        o_ref[:, pl.ds(j * K, K)] = a3[:, j]


def _transpose_table(emb_t):
    return pl.pallas_call(
        _tr_body,
        out_shape=jax.ShapeDtypeStruct((VPAD * K // 128, 128), jnp.float32),
        grid=(NTB,),
        in_specs=[pl.BlockSpec((K, CW), lambda i: (0, i))],
        out_specs=pl.BlockSpec((CW * K // 128, 128), lambda i: (i, 0)),
    )(emb_t)


BLK = 2048
SROWS = BLK * EMBP // 128


def _dense_body(h_ref, fc_ref, linw_ref, linb_ref, w1_ref, b1_ref,
                w2_ref, b2_ref, w3_ref, b3_ref, y_ref):
    s4 = h_ref[...].reshape(BLK, 4, 128)

    s = jnp.zeros((BLK, K), jnp.float32)
    sq = jnp.zeros((BLK, K), jnp.float32)
    a1 = jnp.zeros((BLK, 128), jnp.float32)
    c = lax.broadcasted_iota(jnp.int32, (128, K), 0)
    k = lax.broadcasted_iota(jnp.int32, (128, K), 1)
    match = (c % K) == k
    for ct in range(4):
        valid = (ct * 8 + c // K) < F
        sel = jnp.where(match & valid, 1.0, 0.0)
        hct = s4[:, ct]
        s = s + jnp.dot(hct, sel, preferred_element_type=jnp.float32)
        sq = sq + jnp.dot(hct * hct, sel, preferred_element_type=jnp.float32)
        a1 = a1 + jnp.dot(hct, w1_ref[pl.ds(ct * 128, 128)],
                          preferred_element_type=jnp.float32)
    inter = jnp.sum(s * s - sq, axis=1, keepdims=True)

    tot = jnp.sum(fc_ref[...])
    linw = jnp.sum(linw_ref[...])
    linb = jnp.sum(linb_ref[...])
    linear_term = linw * tot + jnp.float32(TOTAL) * linb

    h1 = jnp.maximum(a1 + b1_ref[...], 0.0)
    h2 = jnp.maximum(
        jnp.dot(h1, w2_ref[...], preferred_element_type=jnp.float32)
        + b2_ref[...], 0.0)
    m = jnp.dot(h2, w3_ref[...], preferred_element_type=jnp.float32) + b3_ref[...]

    z = linear_term + 0.5 * inter + m
    y_ref[...] = 1.0 / (1.0 + jnp.exp(-z))


def _dense(hslab, fcvals, lin_W, lin_b, W1p, b1, W2, b2, W3, b3):
    grid = (B // BLK,)
    zero = lambda i: (0, 0)
    return pl.pallas_call(
        _dense_body,
        out_shape=jax.ShapeDtypeStruct((B, 1), jnp.float32),
        grid=grid,
        in_specs=[
            pl.BlockSpec((SROWS, 128), lambda i: (i, 0)),
            pl.BlockSpec((TOTAL // CHUNK, CHUNK), zero),
            pl.BlockSpec((1, 1), zero),
            pl.BlockSpec((1, 1), zero),
            pl.BlockSpec((EMBP, 128), zero),
            pl.BlockSpec((1, 128), zero),
            pl.BlockSpec((128, 64), zero),
            pl.BlockSpec((1, 64), zero),
            pl.BlockSpec((64, 1), zero),
            pl.BlockSpec((1, 1), zero),
        ],
        out_specs=pl.BlockSpec((BLK, 1), lambda i: (i, 0)),
    )(hslab, fcvals, lin_W, lin_b, W1p, b1, W2, b2, W3, b3)


def kernel(x, fc_table, emb_table, lin_W, lin_b, W1, b1, W2, b2, W3, b3):
    xp = jnp.concatenate([x, x[:, :FP - F]], axis=1)
    xperm = xp.reshape(TOTE // CHUNK, CHUNK)
    xflat = x.reshape(TOTAL // CHUNK, CHUNK)
    fc_flat = fc_table.reshape(-1)
    W1p = jnp.pad(W1, ((0, EMBP - F * K), (0, 0)))
    emb_lin = _transpose_table(emb_table.T).reshape(VPAD, K)
    rows, fcvals = _sc_gather(xperm, xflat, emb_lin, fc_flat)
    hslab = rows.reshape(TOTE * K // 128, 128)
    return _dense(hslab, fcvals, lin_W, lin_b.reshape(1, 1),
                  W1p, b1.reshape(1, -1), W2, b2.reshape(1, -1),
                  W3, b3.reshape(1, 1))
